# Initial kernel scaffold; baseline (speedup 1.0000x reference)
import functools

import jax
import jax.numpy as jnp
from jax import lax
from jax.experimental import pallas as pl
from jax.experimental.pallas import tpu as pltpu
from jax.experimental.pallas import tpu_sc as plsc

N = 10000
E = 320000
D = 128
H = 128
HH = 64

NC = 2
NS = 16
E_PER_TILE = E // NS
CHUNK = 80
NCHUNK = E_PER_TILE // CHUNK
ROWS_PER_TILE = N // NS
ZROWS = 125


def _node_proj(h, W, b, relu_in):
    BN = 2000

    def body(h_ref, w_ref, b_ref, o_ref):
        hh = h_ref[...]
        if relu_in:
            hh = jnp.maximum(hh, 0.0)
        o_ref[0] = jnp.dot(hh, w_ref[...],
                           preferred_element_type=jnp.float32) + b_ref[...]

    return pl.pallas_call(
        body,
        grid=(NC, N // BN),
        in_specs=[
            pl.BlockSpec((BN, D), lambda c, i: (i, 0)),
            pl.BlockSpec((D, HH), lambda c, i: (0, c)),
            pl.BlockSpec((1, HH), lambda c, i: (0, c)),
        ],
        out_specs=pl.BlockSpec((1, BN, HH), lambda c, i: (c, i, 0)),
        out_shape=jax.ShapeDtypeStruct((NC, N, HH), jnp.float32),
    )(h, W, b.reshape(1, H))


def _edge_proj(ea, W, b):
    BE = 8000
    DE = ea.shape[1]

    def body(a_ref, w_ref, b_ref, o_ref):
        o_ref[0] = jnp.dot(a_ref[...], w_ref[...],
                           preferred_element_type=jnp.float32) + b_ref[...]

    return pl.pallas_call(
        body,
        grid=(NC, E // BE),
        in_specs=[
            pl.BlockSpec((BE, DE), lambda c, i: (i, 0)),
            pl.BlockSpec((DE, HH), lambda c, i: (0, c)),
            pl.BlockSpec((1, HH), lambda c, i: (0, c)),
        ],
        out_specs=pl.BlockSpec((1, BE, HH), lambda c, i: (c, i, 0)),
        out_shape=jax.ShapeDtypeStruct((NC, E, HH), jnp.float32),
    )(ea, W, b.reshape(1, H))


def _final_proj(h, Wo, bo):
    BN = 2000

    def body(h_ref, w_ref, b_ref, o_ref):
        o_ref[...] = jnp.dot(jnp.maximum(h_ref[...], 0.0), w_ref[...],
                             preferred_element_type=jnp.float32) + b_ref[...]

    return pl.pallas_call(
        body,
        grid=(N // BN,),
        in_specs=[
            pl.BlockSpec((BN, H), lambda i: (i, 0)),
            pl.BlockSpec((H, 1), lambda i: (0, 0)),
            pl.BlockSpec((1, 1), lambda i: (0, 0)),
        ],
        out_specs=pl.BlockSpec((BN, 1), lambda i: (i, 0)),
        out_shape=jax.ShapeDtypeStruct((N, 1), jnp.float32),
    )(h, Wo, bo.reshape(1, 1))


def _sc_message_pass(xn2, ea2, src, dst):
    mesh = plsc.VectorSubcoreMesh(core_axis_name="c", subcore_axis_name="s")

    @functools.partial(
        pl.kernel,
        out_type=jax.ShapeDtypeStruct((NC * N, HH), jnp.float32),
        mesh=mesh,
        scratch_types=[
            pltpu.VMEM_SHARED((N, HH), jnp.float32),
            pltpu.VMEM_SHARED((N, HH), jnp.float32),
            pltpu.VMEM((ZROWS, HH), jnp.float32),
            pltpu.VMEM((CHUNK,), jnp.int32),
            pltpu.VMEM((CHUNK,), jnp.int32),
            pltpu.VMEM((CHUNK, HH), jnp.float32),
            pltpu.VMEM((CHUNK, HH), jnp.float32),
            pltpu.SemaphoreType.DMA,
        ],
    )
    def k(xn_hbm, ea_hbm, src_hbm, dst_hbm, out_hbm,
          spm_xn, spm_acc, vz, gi, di, gb, eb, sem):
        c = lax.axis_index("c")
        s = lax.axis_index("s")

        @pl.when(s == 0)
        def _():
            pltpu.sync_copy(xn_hbm.at[pl.ds(c * N, N)], spm_xn)

        @pl.loop(0, ZROWS)
        def _(r):
            for q in range(HH // 16):
                vz[r, pl.ds(q * 16, 16)] = jnp.zeros((16,), jnp.float32)

        for rep in range(ROWS_PER_TILE // ZROWS):
            pltpu.sync_copy(
                vz, spm_acc.at[pl.ds(s * ROWS_PER_TILE + rep * ZROWS, ZROWS)])

        plsc.subcore_barrier()

        tile_base = s * E_PER_TILE

        @pl.loop(0, NCHUNK)
        def _(j):
            base = tile_base + j * CHUNK
            pltpu.sync_copy(src_hbm.at[pl.ds(base, CHUNK)], gi)
            pltpu.sync_copy(dst_hbm.at[pl.ds(base, CHUNK)], di)
            pltpu.sync_copy(ea_hbm.at[pl.ds(c * E + base, CHUNK)], eb)
            pltpu.async_copy(spm_xn.at[gi], gb, sem).wait()

            @pl.loop(0, CHUNK)
            def _(r):
                for q in range(HH // 16):
                    sl = pl.ds(q * 16, 16)
                    gb[r, sl] = jnp.maximum(gb[r, sl] + eb[r, sl], 0.0)

            pltpu.sync_copy(gb, spm_acc.at[di], add=True)

        plsc.subcore_barrier()

        pltpu.sync_copy(
            spm_acc.at[pl.ds(s * ROWS_PER_TILE, ROWS_PER_TILE)],
            out_hbm.at[pl.ds(c * N + s * ROWS_PER_TILE, ROWS_PER_TILE)])

    return k(xn2, ea2, src, dst)


def _layer(h, edge_attr, src, dst, Wn, bn, We, be, relu_in):
    xn = _node_proj(h, Wn, bn, relu_in).reshape(NC * N, HH)
    ea = _edge_proj(edge_attr, We, be).reshape(NC * E, HH)
    o = _sc_message_pass(xn, ea, src, dst).reshape(NC, N, HH)
    return jnp.concatenate([o[0], o[1]], axis=1)


def kernel(x, edge_index, edge_attr,
           Wn1, bn1, We1, be1,
           Wn2, bn2, We2, be2,
           Wn3, bn3, We3, be3,
           Wo, bo):
    src = edge_index[0]
    dst = edge_index[1]
    h = _layer(x, edge_attr, src, dst, Wn1, bn1, We1, be1, relu_in=False)
    h = _layer(h, edge_attr, src, dst, Wn2, bn2, We2, be2, relu_in=True)
    h = _layer(h, edge_attr, src, dst, Wn3, bn3, We3, be3, relu_in=True)
    out = _final_proj(h, Wo, bo)
    return out.reshape(N)


# speedup vs baseline: 2.5189x; 2.5189x over previous
import functools

import jax
import jax.numpy as jnp
from jax import lax
from jax.experimental import pallas as pl
from jax.experimental.pallas import tpu as pltpu
from jax.experimental.pallas import tpu_sc as plsc

N = 10000
E = 320000
D = 128
H = 128

NC = 2
NS = 16
NW = NC * NS
E_PER_TILE = E // NW
CHUNK = 80
NCHUNK = E_PER_TILE // CHUNK
NP = 10240
ROWS_PER_TILE = NP // NS
ZROWS = 128


def _node_proj1(h, W, b):
    BN = 2000

    def body(h_ref, w_ref, b_ref, o_ref):
        o_ref[...] = jnp.dot(h_ref[...], w_ref[...],
                             preferred_element_type=jnp.float32) + b_ref[...]

    return pl.pallas_call(
        body,
        grid=(N // BN,),
        in_specs=[
            pl.BlockSpec((BN, D), lambda i: (i, 0)),
            pl.BlockSpec((D, H), lambda i: (0, 0)),
            pl.BlockSpec((1, H), lambda i: (0, 0)),
        ],
        out_specs=pl.BlockSpec((BN, H), lambda i: (i, 0)),
        out_shape=jax.ShapeDtypeStruct((N, H), jnp.float32),
    )(h, W, b.reshape(1, H))


def _node_proj2(acc, W, b):
    BN = 2048

    def body(a0_ref, a1_ref, w_ref, b_ref, o_ref):
        hh = jnp.maximum(a0_ref[...] + a1_ref[...], 0.0)
        o_ref[...] = jnp.dot(hh, w_ref[...],
                             preferred_element_type=jnp.float32) + b_ref[...]

    nblk = NP // BN
    return pl.pallas_call(
        body,
        grid=(nblk,),
        in_specs=[
            pl.BlockSpec((BN, H), lambda i: (i, 0)),
            pl.BlockSpec((BN, H), lambda i, _n=nblk: (_n + i, 0)),
            pl.BlockSpec((D, H), lambda i: (0, 0)),
            pl.BlockSpec((1, H), lambda i: (0, 0)),
        ],
        out_specs=pl.BlockSpec((BN, H), lambda i: (i, 0)),
        out_shape=jax.ShapeDtypeStruct((NP, H), jnp.float32),
    )(acc, acc, W, b.reshape(1, H))


def _edge_proj(ea, W, b):
    BE = 8000
    DE = ea.shape[1]

    def body(a_ref, w_ref, b_ref, o_ref):
        o_ref[...] = jnp.dot(a_ref[...], w_ref[...],
                             preferred_element_type=jnp.float32) + b_ref[...]

    return pl.pallas_call(
        body,
        grid=(E // BE,),
        in_specs=[
            pl.BlockSpec((BE, DE), lambda i: (i, 0)),
            pl.BlockSpec((DE, H), lambda i: (0, 0)),
            pl.BlockSpec((1, H), lambda i: (0, 0)),
        ],
        out_specs=pl.BlockSpec((BE, H), lambda i: (i, 0)),
        out_shape=jax.ShapeDtypeStruct((E, H), jnp.float32),
    )(ea, W, b.reshape(1, H))


def _final_proj(acc, Wo, bo):
    BN = 2048

    def body(a0_ref, a1_ref, w_ref, b_ref, o_ref):
        hh = jnp.maximum(a0_ref[...] + a1_ref[...], 0.0)
        o_ref[...] = jnp.dot(hh, w_ref[...],
                             preferred_element_type=jnp.float32) + b_ref[...]

    nblk = NP // BN
    return pl.pallas_call(
        body,
        grid=(nblk,),
        in_specs=[
            pl.BlockSpec((BN, H), lambda i: (i, 0)),
            pl.BlockSpec((BN, H), lambda i, _n=nblk: (_n + i, 0)),
            pl.BlockSpec((H, 1), lambda i: (0, 0)),
            pl.BlockSpec((1, 1), lambda i: (0, 0)),
        ],
        out_specs=pl.BlockSpec((BN, 1), lambda i: (i, 0)),
        out_shape=jax.ShapeDtypeStruct((NP, 1), jnp.float32),
    )(acc, acc, Wo, bo.reshape(1, 1))


def _sc_message_pass(xn, ea, src, dst):
    mesh = plsc.VectorSubcoreMesh(core_axis_name="c", subcore_axis_name="s")

    @functools.partial(
        pl.kernel,
        out_type=jax.ShapeDtypeStruct((NC * NP, H), jnp.float32),
        mesh=mesh,
        scratch_types=[
            pltpu.VMEM_SHARED((NP, H), jnp.float32),
            pltpu.VMEM((ZROWS, H), jnp.float32),
            pltpu.VMEM((CHUNK,), jnp.int32),
            pltpu.VMEM((CHUNK,), jnp.int32),
            pltpu.VMEM((CHUNK, H), jnp.float32),
            pltpu.VMEM((CHUNK, H), jnp.float32),
            pltpu.SemaphoreType.DMA,
        ],
    )
    def k(xn_hbm, ea_hbm, src_hbm, dst_hbm, out_hbm,
          spm_acc, vz, gi, di, gb, eb, sem):
        c = lax.axis_index("c")
        s = lax.axis_index("s")

        @pl.loop(0, ZROWS)
        def _(r):
            for q in range(H // 16):
                vz[r, pl.ds(q * 16, 16)] = jnp.zeros((16,), jnp.float32)

        for rep in range(ROWS_PER_TILE // ZROWS):
            pltpu.sync_copy(
                vz, spm_acc.at[pl.ds(s * ROWS_PER_TILE + rep * ZROWS, ZROWS)])

        plsc.subcore_barrier()

        tile_base = (c * NS + s) * E_PER_TILE

        @pl.loop(0, NCHUNK)
        def _(j):
            base = tile_base + j * CHUNK
            pltpu.sync_copy(src_hbm.at[pl.ds(base, CHUNK)], gi)
            pltpu.sync_copy(dst_hbm.at[pl.ds(base, CHUNK)], di)
            pltpu.sync_copy(ea_hbm.at[pl.ds(base, CHUNK)], eb)
            pltpu.async_copy(xn_hbm.at[gi], gb, sem).wait()

            @pl.loop(0, CHUNK)
            def _(r):
                for q in range(H // 16):
                    sl = pl.ds(q * 16, 16)
                    gb[r, sl] = jnp.maximum(gb[r, sl] + eb[r, sl], 0.0)

            pltpu.sync_copy(gb, spm_acc.at[di], add=True)

        plsc.subcore_barrier()

        pltpu.sync_copy(
            spm_acc.at[pl.ds(s * ROWS_PER_TILE, ROWS_PER_TILE)],
            out_hbm.at[pl.ds(c * NP + s * ROWS_PER_TILE, ROWS_PER_TILE)])

    return k(xn, ea, src, dst)


def kernel(x, edge_index, edge_attr,
           Wn1, bn1, We1, be1,
           Wn2, bn2, We2, be2,
           Wn3, bn3, We3, be3,
           Wo, bo):
    src = edge_index[0]
    dst = edge_index[1]

    xn = _node_proj1(x, Wn1, bn1)
    ea = _edge_proj(edge_attr, We1, be1)
    acc = _sc_message_pass(xn, ea, src, dst)

    xn = _node_proj2(acc, Wn2, bn2)
    ea = _edge_proj(edge_attr, We2, be2)
    acc = _sc_message_pass(xn, ea, src, dst)

    xn = _node_proj2(acc, Wn3, bn3)
    ea = _edge_proj(edge_attr, We3, be3)
    acc = _sc_message_pass(xn, ea, src, dst)

    out = _final_proj(acc, Wo, bo)
    return out.reshape(NP)[:N]
